# baseline (speedup 1.0000x reference)
import functools

import jax
import jax.numpy as jnp
from jax import lax
from jax.experimental import pallas as pl
from jax.experimental.pallas import tpu as pltpu
from jax.experimental.pallas import tpu_sc as plsc

N_NODES = 100000
N_EDGES = 3200000
D = 16

NC = 2
NS = 16
NW = NC * NS

GRP = 128
CHUNK = 1024
GPC = CHUNK // GRP
FULL_CHUNKS = N_EDGES // CHUNK
BASE_CHUNKS = FULL_CHUNKS // NW
EXTRA = FULL_CHUNKS - BASE_CHUNKS * NW
N_NODES_PAD = 100096
ROWS_PER_TILE = N_NODES_PAD // NS


def _sc_body(mj_hbm, dst_hbm, out_hbm, idx_v, rows_v, accum):
    c = lax.axis_index("c")
    s = lax.axis_index("s")
    w = c * NS + s

    def zero_rows(i, _):
        rows_v[i, :] = jnp.zeros((D,), jnp.float32)
        return _

    lax.fori_loop(0, CHUNK, zero_rows, None)
    base_row = s * ROWS_PER_TILE
    for k in range(ROWS_PER_TILE // CHUNK):
        pltpu.sync_copy(
            rows_v.at[pl.ds(0, CHUNK)],
            accum.at[pl.ds(base_row + k * CHUNK, CHUNK)],
        )
    zrem = ROWS_PER_TILE % CHUNK
    pltpu.sync_copy(
        rows_v.at[pl.ds(0, zrem)],
        accum.at[pl.ds(base_row + (ROWS_PER_TILE // CHUNK) * CHUNK, zrem)],
    )
    plsc.subcore_barrier()

    n_chunks = BASE_CHUNKS + jnp.where(w < EXTRA, 1, 0)
    start_chunk = BASE_CHUNKS * w + jnp.minimum(w, EXTRA)

    def chunk_body(i, _):
        ck = start_chunk + i
        pltpu.sync_copy(dst_hbm.at[pl.ds(ck * CHUNK, CHUNK)], idx_v)
        pltpu.sync_copy(mj_hbm.at[pl.ds(ck * CHUNK, CHUNK)], rows_v)
        for j in range(GPC):
            pltpu.sync_copy(
                rows_v.at[pl.ds(j * GRP, GRP)],
                accum.at[idx_v.at[pl.ds(j * GRP, GRP)]],
                add=True,
            )
        return _

    lax.fori_loop(0, n_chunks, chunk_body, None)

    plsc.subcore_barrier()

    pltpu.sync_copy(
        accum.at[pl.ds(s * ROWS_PER_TILE, ROWS_PER_TILE)],
        out_hbm.at[pl.ds(c * N_NODES_PAD + s * ROWS_PER_TILE, ROWS_PER_TILE)],
    )


@jax.jit
def _sc_scatter(mj, dst_groups):
    mesh = plsc.VectorSubcoreMesh(core_axis_name="c", subcore_axis_name="s")
    return pl.kernel(
        _sc_body,
        out_type=jax.ShapeDtypeStruct((NC * N_NODES_PAD, D), jnp.float32),
        mesh=mesh,
        compiler_params=pltpu.CompilerParams(use_tc_tiling_on_sc=False),
        scratch_types=[
            pltpu.VMEM((CHUNK,), jnp.int32),
            pltpu.VMEM((CHUNK, D), jnp.float32),
            pltpu.VMEM_SHARED((N_NODES_PAD, D), jnp.float32),
        ],
    )(mj, dst_groups)


def _add_body(a_ref, o_ref):
    o_ref[...] = a_ref[0] + a_ref[1]


@jax.jit
def _combine(partials):
    rows128 = NC * N_NODES_PAD * D // (NC * GRP)
    p2 = partials.reshape(NC, rows128, GRP)
    out = pl.pallas_call(
        _add_body,
        out_shape=jax.ShapeDtypeStruct((rows128, GRP), jnp.float32),
    )(p2)
    return out.reshape(N_NODES_PAD, D)[:N_NODES]


def kernel(mj, edge_index):
    dst = edge_index[1, :].astype(jnp.int32)
    partials = _sc_scatter(mj, dst)
    return _combine(partials)


# speedup vs baseline: 1.0320x; 1.0320x over previous
import functools

import jax
import jax.numpy as jnp
from jax import lax
from jax.experimental import pallas as pl
from jax.experimental.pallas import tpu as pltpu
from jax.experimental.pallas import tpu_sc as plsc

N_NODES = 100000
N_EDGES = 3200000
D = 16

NC = 2
NS = 16
NW = NC * NS

GRP = 128
CHUNK = 1024
GPC = CHUNK // GRP
FULL_CHUNKS = N_EDGES // CHUNK
BASE_CHUNKS = FULL_CHUNKS // NW
EXTRA = FULL_CHUNKS - BASE_CHUNKS * NW
N_NODES_PAD = 100096
ROWS_PER_TILE = N_NODES_PAD // NS
ZROWS = 391


def _sc_body(mj_hbm, dst_hbm, out_hbm, idx_v, rows_v, zero_v, accum):
    c = lax.axis_index("c")
    s = lax.axis_index("s")
    w = c * NS + s

    def zero_rows(i, _):
        zero_v[i, :] = jnp.zeros((D,), jnp.float32)
        return _

    lax.fori_loop(0, ZROWS, zero_rows, None)
    base_row = s * ROWS_PER_TILE
    for k in range(ROWS_PER_TILE // ZROWS):
        pltpu.sync_copy(
            zero_v,
            accum.at[pl.ds(base_row + k * ZROWS, ZROWS)],
        )
    plsc.subcore_barrier()

    n_chunks = BASE_CHUNKS + jnp.where(w < EXTRA, 1, 0)
    start_chunk = BASE_CHUNKS * w + jnp.minimum(w, EXTRA)

    def chunk_body(i, _):
        ck = start_chunk + i
        pltpu.sync_copy(dst_hbm.at[pl.ds(ck * CHUNK, CHUNK)], idx_v)
        pltpu.sync_copy(mj_hbm.at[pl.ds(ck * CHUNK, CHUNK)], rows_v)
        pltpu.sync_copy(rows_v, accum.at[idx_v], add=True)
        return _

    lax.fori_loop(0, n_chunks, chunk_body, None)

    plsc.subcore_barrier()

    pltpu.sync_copy(
        accum.at[pl.ds(s * ROWS_PER_TILE, ROWS_PER_TILE)],
        out_hbm.at[pl.ds(c * N_NODES_PAD + s * ROWS_PER_TILE, ROWS_PER_TILE)],
    )


@jax.jit
def _sc_scatter(mj, dst_groups):
    mesh = plsc.VectorSubcoreMesh(core_axis_name="c", subcore_axis_name="s")
    return pl.kernel(
        _sc_body,
        out_type=jax.ShapeDtypeStruct((NC * N_NODES_PAD, D), jnp.float32),
        mesh=mesh,
        compiler_params=pltpu.CompilerParams(use_tc_tiling_on_sc=False),
        scratch_types=[
            pltpu.VMEM((CHUNK,), jnp.int32),
            pltpu.VMEM((CHUNK, D), jnp.float32),
            pltpu.VMEM((ZROWS, D), jnp.float32),
            pltpu.VMEM_SHARED((N_NODES_PAD, D), jnp.float32),
        ],
    )(mj, dst_groups)


def _add_body(a_ref, o_ref):
    o_ref[...] = a_ref[0] + a_ref[1]


@jax.jit
def _combine(partials):
    rows128 = NC * N_NODES_PAD * D // (NC * GRP)
    p2 = partials.reshape(NC, rows128, GRP)
    out = pl.pallas_call(
        _add_body,
        out_shape=jax.ShapeDtypeStruct((rows128, GRP), jnp.float32),
    )(p2)
    return out.reshape(N_NODES_PAD, D)[:N_NODES]


def kernel(mj, edge_index):
    dst = edge_index[1, :].astype(jnp.int32)
    partials = _sc_scatter(mj, dst)
    return _combine(partials)


# speedup vs baseline: 1.5798x; 1.5308x over previous
import functools

import jax
import jax.numpy as jnp
from jax import lax
from jax.experimental import pallas as pl
from jax.experimental.pallas import tpu as pltpu
from jax.experimental.pallas import tpu_sc as plsc

N_NODES = 100000
N_EDGES = 3200000
D = 16

NC = 2
NS = 16
NW = NC * NS

CHUNK = 1024
FULL_CHUNKS = N_EDGES // CHUNK
BASE_CHUNKS = FULL_CHUNKS // NW
EXTRA = FULL_CHUNKS - BASE_CHUNKS * NW
N_NODES_PAD = 100096
COLS_PER_TILE = N_NODES_PAD // NS


def _sc_body(mjt_hbm, dst_hbm, out_hbm, idx_v, feat_v, accum):
    c = lax.axis_index("c")
    s = lax.axis_index("s")
    w = c * NS + s

    def zero_row(i, _):
        feat_v[0, pl.ds(i * D, D)] = jnp.zeros((D,), jnp.float32)
        return _

    lax.fori_loop(0, CHUNK // D, zero_row, None)
    base_col = s * COLS_PER_TILE
    for d in range(D):
        for k in range(COLS_PER_TILE // CHUNK):
            pltpu.sync_copy(
                feat_v.at[0],
                accum.at[d, pl.ds(base_col + k * CHUNK, CHUNK)],
            )
        zrem = COLS_PER_TILE % CHUNK
        pltpu.sync_copy(
            feat_v.at[0, pl.ds(0, zrem)],
            accum.at[d, pl.ds(base_col + (COLS_PER_TILE // CHUNK) * CHUNK, zrem)],
        )
    plsc.subcore_barrier()

    n_chunks = BASE_CHUNKS + jnp.where(w < EXTRA, 1, 0)
    start_chunk = BASE_CHUNKS * w + jnp.minimum(w, EXTRA)

    def chunk_body(i, _):
        ck = start_chunk + i
        pltpu.sync_copy(dst_hbm.at[pl.ds(ck * CHUNK, CHUNK)], idx_v)
        pltpu.sync_copy(mjt_hbm.at[:, pl.ds(ck * CHUNK, CHUNK)], feat_v)
        for d in range(D):
            pltpu.sync_copy(
                feat_v.at[d],
                accum.at[d].at[idx_v],
                add=True,
            )
        return _

    lax.fori_loop(0, n_chunks, chunk_body, None)

    plsc.subcore_barrier()

    pltpu.sync_copy(
        accum.at[:, pl.ds(base_col, COLS_PER_TILE)],
        out_hbm.at[pl.ds(c * D, D), pl.ds(base_col, COLS_PER_TILE)],
    )


@jax.jit
def _sc_scatter(mjt, dst):
    mesh = plsc.VectorSubcoreMesh(core_axis_name="c", subcore_axis_name="s")
    return pl.kernel(
        _sc_body,
        out_type=jax.ShapeDtypeStruct((NC * D, N_NODES_PAD), jnp.float32),
        mesh=mesh,
        compiler_params=pltpu.CompilerParams(use_tc_tiling_on_sc=False),
        scratch_types=[
            pltpu.VMEM((CHUNK,), jnp.int32),
            pltpu.VMEM((D, CHUNK), jnp.float32),
            pltpu.VMEM_SHARED((D, N_NODES_PAD), jnp.float32),
        ],
    )(mjt, dst)


def _add_body(a_ref, o_ref):
    o_ref[...] = a_ref[0] + a_ref[1]


@jax.jit
def _combine(partials):
    rows128 = D * N_NODES_PAD // 128
    p2 = partials.reshape(NC, rows128, 128)
    out = pl.pallas_call(
        _add_body,
        out_shape=jax.ShapeDtypeStruct((rows128, 128), jnp.float32),
    )(p2)
    return out.reshape(D, N_NODES_PAD)[:, :N_NODES].T


def kernel(mj, edge_index):
    dst = edge_index[1, :].astype(jnp.int32)
    partials = _sc_scatter(mj.T, dst)
    return _combine(partials)


# speedup vs baseline: 1.7520x; 1.1090x over previous
import functools

import jax
import jax.numpy as jnp
from jax import lax
from jax.experimental import pallas as pl
from jax.experimental.pallas import tpu as pltpu
from jax.experimental.pallas import tpu_sc as plsc

N_NODES = 100000
N_EDGES = 3200000
D = 16

NC = 2
NS = 16
NW = NC * NS

CHUNK = 1024
FULL_CHUNKS = N_EDGES // CHUNK
BASE_CHUNKS = FULL_CHUNKS // NW
EXTRA = FULL_CHUNKS - BASE_CHUNKS * NW
N_NODES_PAD = 100096
COLS_PER_TILE = N_NODES_PAD // NS


def _sc_body(mjt_hbm, dst_hbm, out_hbm, idx_v, feat_v, accum, sem):
    c = lax.axis_index("c")
    s = lax.axis_index("s")
    w = c * NS + s

    def zero_row(i, _):
        feat_v[0, pl.ds(i * D, D)] = jnp.zeros((D,), jnp.float32)
        return _

    lax.fori_loop(0, CHUNK // D, zero_row, None)
    base_col = s * COLS_PER_TILE
    for d in range(D):
        for k in range(COLS_PER_TILE // CHUNK):
            pltpu.sync_copy(
                feat_v.at[0],
                accum.at[d, pl.ds(base_col + k * CHUNK, CHUNK)],
            )
        zrem = COLS_PER_TILE % CHUNK
        pltpu.sync_copy(
            feat_v.at[0, pl.ds(0, zrem)],
            accum.at[d, pl.ds(base_col + (COLS_PER_TILE // CHUNK) * CHUNK, zrem)],
        )
    plsc.subcore_barrier()

    n_chunks = BASE_CHUNKS + jnp.where(w < EXTRA, 1, 0)
    start_chunk = BASE_CHUNKS * w + jnp.minimum(w, EXTRA)

    def chunk_body(i, _):
        ck = start_chunk + i
        pltpu.sync_copy(dst_hbm.at[pl.ds(ck * CHUNK, CHUNK)], idx_v)
        pltpu.sync_copy(mjt_hbm.at[:, pl.ds(ck * CHUNK, CHUNK)], feat_v)
        descs = [
            pltpu.async_copy(feat_v.at[d], accum.at[d].at[idx_v], sem, add=True)
            for d in range(D)
        ]
        for desc in descs:
            desc.wait()
        return _

    lax.fori_loop(0, n_chunks, chunk_body, None)

    plsc.subcore_barrier()

    pltpu.sync_copy(
        accum.at[:, pl.ds(base_col, COLS_PER_TILE)],
        out_hbm.at[pl.ds(c * D, D), pl.ds(base_col, COLS_PER_TILE)],
    )


@jax.jit
def _sc_scatter(mjt, dst):
    mesh = plsc.VectorSubcoreMesh(core_axis_name="c", subcore_axis_name="s")
    return pl.kernel(
        _sc_body,
        out_type=jax.ShapeDtypeStruct((NC * D, N_NODES_PAD), jnp.float32),
        mesh=mesh,
        compiler_params=pltpu.CompilerParams(use_tc_tiling_on_sc=False),
        scratch_types=[
            pltpu.VMEM((CHUNK,), jnp.int32),
            pltpu.VMEM((D, CHUNK), jnp.float32),
            pltpu.VMEM_SHARED((D, N_NODES_PAD), jnp.float32),
            pltpu.SemaphoreType.DMA,
        ],
    )(mjt, dst)


def _add_body(a_ref, o_ref):
    o_ref[...] = a_ref[0] + a_ref[1]


@jax.jit
def _combine(partials):
    rows128 = D * N_NODES_PAD // 128
    p2 = partials.reshape(NC, rows128, 128)
    out = pl.pallas_call(
        _add_body,
        out_shape=jax.ShapeDtypeStruct((rows128, 128), jnp.float32),
    )(p2)
    return out.reshape(D, N_NODES_PAD)[:, :N_NODES].T


def kernel(mj, edge_index):
    dst = edge_index[1, :].astype(jnp.int32)
    partials = _sc_scatter(mj.T, dst)
    return _combine(partials)


# speedup vs baseline: 2.1394x; 1.2211x over previous
import functools

import jax
import jax.numpy as jnp
from jax import lax
from jax.experimental import pallas as pl
from jax.experimental.pallas import tpu as pltpu
from jax.experimental.pallas import tpu_sc as plsc

N_NODES = 100000
N_EDGES = 3200000
D = 16

NC = 2
NS = 16
NW = NC * NS

CHUNK = 800
FULL_CHUNKS = N_EDGES // CHUNK
BASE_CHUNKS = FULL_CHUNKS // NW
N_NODES_PAD = 100096
COLS_PER_TILE = N_NODES_PAD // NS


def _sc_body(mjt_hbm, dst_hbm, out_hbm, idx2, feat2, accum, sem_in, sem_sc):
    c = lax.axis_index("c")
    s = lax.axis_index("s")
    w = c * NS + s

    def zero_row(i, _):
        feat2[0, 0, pl.ds(i * D, D)] = jnp.zeros((D,), jnp.float32)
        return _

    lax.fori_loop(0, CHUNK // D, zero_row, None)
    base_col = s * COLS_PER_TILE
    for d in range(D):
        for k in range(COLS_PER_TILE // CHUNK):
            pltpu.sync_copy(
                feat2.at[0, 0],
                accum.at[d, pl.ds(base_col + k * CHUNK, CHUNK)],
            )
        zrem = COLS_PER_TILE % CHUNK
        pltpu.sync_copy(
            feat2.at[0, 0, pl.ds(0, zrem)],
            accum.at[d, pl.ds(base_col + (COLS_PER_TILE // CHUNK) * CHUNK, zrem)],
        )
    plsc.subcore_barrier()

    start_chunk = BASE_CHUNKS * w

    def load(i, b):
        ck = start_chunk + i
        pltpu.async_copy(dst_hbm.at[pl.ds(ck * CHUNK, CHUNK)], idx2.at[b], sem_in)
        pltpu.async_copy(mjt_hbm.at[:, pl.ds(ck * CHUNK, CHUNK)], feat2.at[b], sem_in)

    def wait_load(i, b):
        ck = start_chunk + i
        pltpu.make_async_copy(
            dst_hbm.at[pl.ds(ck * CHUNK, CHUNK)], idx2.at[b], sem_in
        ).wait()
        pltpu.make_async_copy(
            mjt_hbm.at[:, pl.ds(ck * CHUNK, CHUNK)], feat2.at[b], sem_in
        ).wait()

    load(0, 0)

    def chunk_body(i, _):
        b = lax.rem(i, 2)
        wait_load(i, b)

        @pl.when(i + 1 < BASE_CHUNKS)
        def _():
            load(i + 1, 1 - b)

        descs = [
            pltpu.async_copy(
                feat2.at[b, d], accum.at[d].at[idx2.at[b]], sem_sc, add=True
            )
            for d in range(D)
        ]
        for desc in descs:
            desc.wait()
        return _

    lax.fori_loop(0, BASE_CHUNKS, chunk_body, None)

    plsc.subcore_barrier()

    pltpu.sync_copy(
        accum.at[:, pl.ds(base_col, COLS_PER_TILE)],
        out_hbm.at[pl.ds(c * D, D), pl.ds(base_col, COLS_PER_TILE)],
    )


@jax.jit
def _sc_scatter(mjt, dst):
    mesh = plsc.VectorSubcoreMesh(core_axis_name="c", subcore_axis_name="s")
    return pl.kernel(
        _sc_body,
        out_type=jax.ShapeDtypeStruct((NC * D, N_NODES_PAD), jnp.float32),
        mesh=mesh,
        compiler_params=pltpu.CompilerParams(use_tc_tiling_on_sc=False),
        scratch_types=[
            pltpu.VMEM((2, CHUNK), jnp.int32),
            pltpu.VMEM((2, D, CHUNK), jnp.float32),
            pltpu.VMEM_SHARED((D, N_NODES_PAD), jnp.float32),
            pltpu.SemaphoreType.DMA,
            pltpu.SemaphoreType.DMA,
        ],
    )(mjt, dst)


def _add_body(a_ref, o_ref):
    o_ref[...] = a_ref[0] + a_ref[1]


@jax.jit
def _combine(partials):
    rows128 = D * N_NODES_PAD // 128
    p2 = partials.reshape(NC, rows128, 128)
    out = pl.pallas_call(
        _add_body,
        out_shape=jax.ShapeDtypeStruct((rows128, 128), jnp.float32),
    )(p2)
    return out.reshape(D, N_NODES_PAD)[:, :N_NODES].T


def kernel(mj, edge_index):
    dst = edge_index[1, :].astype(jnp.int32)
    partials = _sc_scatter(mj.T, dst)
    return _combine(partials)
